# baseline (speedup 1.0000x reference)
import jax
import jax.numpy as jnp
from jax import lax
from jax.experimental import pallas as pl
from jax.experimental.pallas import tpu as pltpu
from jax.experimental.pallas import tpu_sc as plsc

NUM_CORES = 2
NUM_SUBCORES = 16
NW = NUM_CORES * NUM_SUBCORES
B = 16384
D = 32
RPB = 128 // D
BPW = B // NW
HALF = BPW // 2
CHUNKS = HALF // 16


def _mf_kernel(uids_hbm, iids_hbm, ut_hbm, it_hbm, wb_hbm, out_hbm,
               idx_u, idx_i, blk_u, blk_i, u_rows, i_rows, wb_v, out_v,
               sem_u, sem_i):
    wid = lax.axis_index("s") * NUM_CORES + lax.axis_index("c")
    base = wid * BPW

    pltpu.sync_copy(wb_hbm, wb_v)
    pltpu.sync_copy(uids_hbm.at[pl.ds(base, BPW)], idx_u)
    pltpu.sync_copy(iids_hbm.at[pl.ds(base, BPW)], idx_i)

    def mk_blocks(j, carry):
        s = pl.ds(j * 16, 16)
        blk_u[s] = idx_u[s] >> 2
        blk_i[s] = idx_i[s] >> 2
        return carry
    lax.fori_loop(0, BPW // 16, mk_blocks, 0)

    w_lo = wb_v[pl.ds(0, 16)]
    w_hi = wb_v[pl.ds(16, 16)]
    bias = wb_v[pl.ds(24, 16)][8]

    def run_half(h):
        cu = pltpu.async_copy(ut_hbm.at[blk_u.at[pl.ds(h * HALF, HALF)]],
                              u_rows, sem_u)
        ci = pltpu.async_copy(it_hbm.at[blk_i.at[pl.ds(h * HALF, HALF)]],
                              i_rows, sem_i)
        cu.wait()
        ci.wait()

        def body(c, carry):
            rows = c * 16 + lax.iota(jnp.int32, 16)
            s = pl.ds(h * HALF + c * 16, 16)
            sub_u = (idx_u[s] & (RPB - 1)) * D
            sub_i = (idx_i[s] & (RPB - 1)) * D
            acc = jnp.full((16,), 0.0, dtype=jnp.float32) + bias
            for d in range(D):
                ucol = plsc.load_gather(u_rows, [rows, sub_u + d])
                icol = plsc.load_gather(i_rows, [rows, sub_i + d])
                w_d = w_lo[d] if d < 16 else w_hi[d - 16]
                acc = acc + ucol * icol * w_d
            z = 1.0 / (1.0 + jnp.exp(-acc))
            out_v[s] = z
            return carry

        lax.fori_loop(0, CHUNKS, body, 0)

    run_half(0)
    run_half(1)
    pltpu.sync_copy(out_v, out_hbm.at[pl.ds(base, BPW)])


@jax.jit
def kernel(input, user_table, item_table, W, b):
    uids = input[:, 0]
    iids = input[:, 1]
    ut4 = user_table.reshape(-1, D * RPB)
    it4 = item_table.reshape(-1, D * RPB)
    wb = jnp.concatenate(
        [W.reshape(-1), b.reshape(-1), jnp.zeros((7,), jnp.float32)])

    mesh = plsc.VectorSubcoreMesh(core_axis_name="c", subcore_axis_name="s")
    run = pl.kernel(
        _mf_kernel,
        mesh=mesh,
        compiler_params=pltpu.CompilerParams(needs_layout_passes=False),
        out_type=jax.ShapeDtypeStruct((B,), jnp.float32),
        scratch_types=[
            pltpu.VMEM((BPW,), jnp.int32),
            pltpu.VMEM((BPW,), jnp.int32),
            pltpu.VMEM((BPW,), jnp.int32),
            pltpu.VMEM((BPW,), jnp.int32),
            pltpu.VMEM((HALF, D * RPB), jnp.float32),
            pltpu.VMEM((HALF, D * RPB), jnp.float32),
            pltpu.VMEM((D + 8,), jnp.float32),
            pltpu.VMEM((BPW,), jnp.float32),
            pltpu.SemaphoreType.DMA,
            pltpu.SemaphoreType.DMA,
        ],
    )
    return run(uids, iids, ut4, it4, wb)
